# baseline (speedup 1.0000x reference)
import functools
import math

import jax
import jax.numpy as jnp
from jax import lax
from jax.experimental import pallas as pl
from jax.experimental.pallas import tpu as pltpu

_V = 100000
_B = 1024
_S = 0.1 / (_V - 2)
_C = 0.9
_ENT = _C * math.log(_C) + (_V - 2) * _S * math.log(_S)

_BR = 32
_NR = _B // _BR
_NFULL = _V // 128
_REM = _V - _NFULL * 128


def _body(t_ref, x_ref, o_ref):
    i = pl.program_id(0)
    t = t_ref[...]
    mask = t != 0
    lane = lax.broadcasted_iota(jnp.int32, (_BR, 128), 1)
    ch0 = x_ref[:, 0:128]
    acc = ch0
    gacc = jnp.where(lane == t, ch0, 0.0)
    for c in range(1, _NFULL):
        ch = x_ref[:, c * 128:(c + 1) * 128]
        acc = acc + ch
        gacc = gacc + jnp.where(lane == t - c * 128, ch, 0.0)
    rs = jnp.sum(acc, axis=1, keepdims=True)
    gv = jnp.sum(gacc, axis=1, keepdims=True)
    if _REM:
        tch = x_ref[:, _NFULL * 128:_V]
        lane_t = lax.broadcasted_iota(jnp.int32, (_BR, _REM), 1)
        rs = rs + jnp.sum(tch, axis=1, keepdims=True)
        gv = gv + jnp.sum(
            jnp.where(lane_t == t - _NFULL * 128, tch, 0.0),
            axis=1, keepdims=True)
    x0 = x_ref[:, 0:1]
    per = jnp.where(mask, _ENT - (_C - _S) * gv - _S * (rs - x0), 0.0)
    partial = jnp.sum(per.astype(jnp.float32))

    @pl.when(i == 0)
    def _init():
        o_ref[0, 0] = partial

    @pl.when(i > 0)
    def _acc():
        o_ref[0, 0] += partial


def _tc_all(t2, x):
    out = pl.pallas_call(
        _body,
        grid=(_NR,),
        in_specs=[
            pl.BlockSpec((_BR, 1), lambda i: (i, 0)),
            pl.BlockSpec((_BR, _V), lambda i: (i, 0)),
        ],
        out_specs=pl.BlockSpec(memory_space=pltpu.SMEM),
        out_shape=jax.ShapeDtypeStruct((1, 1), jnp.float32),
        compiler_params=pltpu.CompilerParams(
            dimension_semantics=("arbitrary",),
        ),
    )(t2, x)
    return out[0, 0]


@jax.jit
def kernel(x, target):
    return _tc_all(target.astype(jnp.int32).reshape(_B, 1), x)


# speedup vs baseline: 1.0040x; 1.0040x over previous
import math

import jax
import jax.numpy as jnp
from jax import lax
from jax.experimental import pallas as pl
from jax.experimental.pallas import tpu as pltpu

_V = 100000
_B = 1024
_S = 0.1 / (_V - 2)
_C = 0.9
_ENT = _C * math.log(_C) + (_V - 2) * _S * math.log(_S)

_BR = 32
_NR = _B // _BR
_NFULL = _V // 128
_REM = _V - _NFULL * 128


def _body(t_ref, x_ref, o_ref):
    i = pl.program_id(0)
    t = t_ref[...]
    mask = t != 0
    lane = lax.broadcasted_iota(jnp.int32, (_BR, 128), 1)
    ch0 = x_ref[:, 0:128]
    acc = ch0
    gacc = jnp.where(lane == t, ch0, 0.0)
    for c in range(1, _NFULL):
        ch = x_ref[:, c * 128:(c + 1) * 128]
        acc = acc + ch
        gacc = gacc + jnp.where(lane == t - c * 128, ch, 0.0)
    rs = jnp.sum(acc, axis=1, keepdims=True)
    gv = jnp.sum(gacc, axis=1, keepdims=True)
    if _REM:
        tch = x_ref[:, _NFULL * 128:_V]
        lane_t = lax.broadcasted_iota(jnp.int32, (_BR, _REM), 1)
        rs = rs + jnp.sum(tch, axis=1, keepdims=True)
        gv = gv + jnp.sum(
            jnp.where(lane_t == t - _NFULL * 128, tch, 0.0),
            axis=1, keepdims=True)
    x0 = x_ref[:, 0:1]
    per = jnp.where(mask, _ENT - (_C - _S) * gv - _S * (rs - x0), 0.0)
    partial = jnp.sum(per.astype(jnp.float32))

    @pl.when(i == 0)
    def _init():
        o_ref[0, 0] = partial

    @pl.when(i > 0)
    def _acc():
        o_ref[0, 0] += partial


def _tc_all(t2, x):
    out = pl.pallas_call(
        _body,
        grid=(_NR,),
        in_specs=[
            pl.BlockSpec((_BR, 1), lambda i: (i, 0)),
            pl.BlockSpec((_BR, _V), lambda i: (i, 0)),
        ],
        out_specs=pl.BlockSpec(memory_space=pltpu.SMEM),
        out_shape=jax.ShapeDtypeStruct((1, 1), jnp.float32),
        compiler_params=pltpu.CompilerParams(
            dimension_semantics=("arbitrary",),
        ),
    )(t2, x)
    return out[0, 0]


@jax.jit
def kernel(x, target):
    return _tc_all(target.astype(jnp.int32).reshape(_B, 1), x)
